# baseline (speedup 1.0000x reference)
import functools

import jax
import jax.numpy as jnp
from jax import lax
from jax.experimental import pallas as pl
from jax.experimental.pallas import tpu as pltpu
from jax.experimental.pallas import tpu_sc as plsc


def _mm_body(emb_ref, w_ref, b_ref, out_ref):
    out_ref[...] = lax.dot_general(
        emb_ref[...], w_ref[...],
        dimension_numbers=(((1,), (1,)), ((), ())),
        preferred_element_type=jnp.float32,
    ) + b_ref[...]


@functools.cache
def _gather_fn(V, D, B, L, Lp):
    info = plsc.get_sparse_core_info()
    nw = info.num_cores * info.num_subcores
    seqs_per_w = B // nw
    idx_per_w = seqs_per_w * Lp
    mesh = plsc.VectorSubcoreMesh(core_axis_name="c", subcore_axis_name="s")

    @functools.partial(
        pl.kernel, mesh=mesh,
        out_type=jax.ShapeDtypeStruct((B, L, D), jnp.float32),
        scratch_types=[
            pltpu.VMEM((idx_per_w,), jnp.int32),
            pltpu.VMEM((Lp, D), jnp.float32),
            pltpu.SemaphoreType.DMA,
        ],
        compiler_params=pltpu.CompilerParams(use_tc_tiling_on_sc=False),
    )
    def k(table_hbm, idx_hbm, out_hbm, idx_v, rows_v, sem):
        wid = lax.axis_index("s") * info.num_cores + lax.axis_index("c")
        pltpu.sync_copy(idx_hbm.at[pl.ds(wid * idx_per_w, idx_per_w)], idx_v)
        seq0 = wid * seqs_per_w

        def body(j, carry):
            pltpu.async_copy(
                table_hbm.at[idx_v.at[pl.ds(j * Lp, Lp)]], rows_v, sem
            ).wait()
            pltpu.sync_copy(rows_v.at[pl.ds(0, L)], out_hbm.at[seq0 + j])
            return carry

        lax.fori_loop(0, seqs_per_w, body, 0)

    return k


def kernel(input_ids, emb_table, lm_head_w, lm_head_b):
    V, H = emb_table.shape
    Vo = lm_head_w.shape[0]
    B, L = input_ids.shape
    Lp = (L + 7) // 8 * 8
    fused = pl.pallas_call(
        _mm_body,
        out_shape=jax.ShapeDtypeStruct((V, Vo), jnp.float32),
    )(emb_table, lm_head_w, lm_head_b.reshape(1, Vo))
    ids_p = jnp.concatenate(
        [input_ids, input_ids[:, : Lp - L]], axis=1
    ).reshape(-1).astype(jnp.int32)
    out = _gather_fn(V, Vo, B, L, Lp)(fused, ids_p)
    return out


# speedup vs baseline: 4.8780x; 4.8780x over previous
import functools

import jax
import jax.numpy as jnp
from jax import lax
from jax.experimental import pallas as pl
from jax.experimental.pallas import tpu as pltpu
from jax.experimental.pallas import tpu_sc as plsc


@functools.cache
def _gather_fn(V, Hp, N):
    info = plsc.get_sparse_core_info()
    nw = info.num_cores * info.num_subcores
    rows_per_w = N // nw
    ch = 80
    nch = rows_per_w // ch
    mesh = plsc.VectorSubcoreMesh(core_axis_name="c", subcore_axis_name="s")

    @functools.partial(
        pl.kernel, mesh=mesh,
        out_type=jax.ShapeDtypeStruct((N, Hp), jnp.float32),
        scratch_types=[
            pltpu.VMEM((rows_per_w,), jnp.int32),
            pltpu.VMEM((ch, Hp), jnp.float32),
            pltpu.SemaphoreType.DMA,
        ],
    )
    def k(table_hbm, idx_hbm, out_hbm, idx_v, rows_v, sem):
        wid = lax.axis_index("s") * info.num_cores + lax.axis_index("c")
        base = wid * rows_per_w
        pltpu.sync_copy(idx_hbm.at[pl.ds(base, rows_per_w)], idx_v)

        def body(c, carry):
            pltpu.async_copy(
                table_hbm.at[idx_v.at[pl.ds(c * ch, ch)]], rows_v, sem
            ).wait()
            pltpu.sync_copy(rows_v, out_hbm.at[pl.ds(base + c * ch, ch)])
            return carry

        lax.fori_loop(0, nch, body, 0)

    return k


def _proj_body(h_ref, w_ref, b_ref, out_ref):
    out_ref[0] = lax.dot_general(
        w_ref[...], h_ref[0],
        dimension_numbers=(((1,), (1,)), ((), ())),
        preferred_element_type=jnp.float32,
    ) + b_ref[...]


def kernel(input_ids, emb_table, lm_head_w, lm_head_b):
    V, H = emb_table.shape
    Vo = lm_head_w.shape[0]
    B, L = input_ids.shape
    Hp = 128
    emb_p = jnp.pad(emb_table, ((0, 0), (0, Hp - H)))
    w_p = jnp.pad(lm_head_w, ((0, 0), (0, Hp - H)))
    ids_t = input_ids.T.reshape(-1).astype(jnp.int32)
    hidden = _gather_fn(V, Hp, B * L)(emb_p, ids_t).reshape(L, B, Hp)
    out_t = pl.pallas_call(
        _proj_body,
        grid=(L,),
        in_specs=[
            pl.BlockSpec((1, B, Hp), lambda l: (l, 0, 0)),
            pl.BlockSpec((Vo, Hp), lambda l: (0, 0)),
            pl.BlockSpec((Vo, 1), lambda l: (0, 0)),
        ],
        out_specs=pl.BlockSpec((1, Vo, B), lambda l: (l, 0, 0)),
        out_shape=jax.ShapeDtypeStruct((L, Vo, B), jnp.float32),
    )(hidden, w_p, lm_head_b.reshape(Vo, 1))
    return jnp.transpose(out_t, (2, 0, 1))


# speedup vs baseline: 5.1024x; 1.0460x over previous
import functools

import jax
import jax.numpy as jnp
from jax import lax
from jax.experimental import pallas as pl
from jax.experimental.pallas import tpu as pltpu
from jax.experimental.pallas import tpu_sc as plsc

_NSPLIT = 2


@functools.cache
def _gather_fn(V, Hp, N):
    info = plsc.get_sparse_core_info()
    nw = info.num_cores * info.num_subcores
    rows_per_w = N // nw
    ch = 80
    nch = rows_per_w // ch
    mesh = plsc.VectorSubcoreMesh(core_axis_name="c", subcore_axis_name="s")

    @functools.partial(
        pl.kernel, mesh=mesh,
        out_type=jax.ShapeDtypeStruct((N, Hp), jnp.float32),
        scratch_types=[
            pltpu.VMEM((rows_per_w,), jnp.int32),
            pltpu.VMEM((ch, Hp), jnp.float32),
            pltpu.SemaphoreType.DMA,
        ],
    )
    def k(table_hbm, idx_hbm, out_hbm, idx_v, rows_v, sem):
        wid = lax.axis_index("s") * info.num_cores + lax.axis_index("c")
        base = wid * rows_per_w
        pltpu.sync_copy(idx_hbm.at[pl.ds(base, rows_per_w)], idx_v)

        def body(c, carry):
            pltpu.async_copy(
                table_hbm.at[idx_v.at[pl.ds(c * ch, ch)]], rows_v, sem
            ).wait()
            pltpu.sync_copy(rows_v, out_hbm.at[pl.ds(base + c * ch, ch)])
            return carry

        lax.fori_loop(0, nch, body, 0)

    return k


def _proj_first(h_ref, w_ref, b_ref, out_ref):
    out_ref[0] = lax.dot_general(
        w_ref[...], h_ref[0],
        dimension_numbers=(((1,), (1,)), ((), ())),
        preferred_element_type=jnp.float32,
    ) + b_ref[...]


def _proj_next(h_ref, w_ref, b_ref, carry_ref, out_ref):
    out_ref[0] = lax.dot_general(
        w_ref[...], h_ref[0],
        dimension_numbers=(((1,), (1,)), ((), ())),
        preferred_element_type=jnp.float32,
    ) + b_ref[...]


def kernel(input_ids, emb_table, lm_head_w, lm_head_b):
    V, H = emb_table.shape
    Vo = lm_head_w.shape[0]
    B, L = input_ids.shape
    Hp = 128
    Lc = L // _NSPLIT
    emb_p = jnp.pad(emb_table, ((0, 0), (0, Hp - H)))
    w_p = jnp.pad(lm_head_w, ((0, 0), (0, Hp - H)))
    bias = lm_head_b.reshape(Vo, 1)
    ids_t = input_ids.T.reshape(-1).astype(jnp.int32)
    gather = _gather_fn(V, Hp, B * Lc)
    out_shape = jax.ShapeDtypeStruct((L, Vo, B), jnp.float32)
    hidden = [
        gather(emb_p, lax.dynamic_slice_in_dim(ids_t, c * B * Lc, B * Lc))
        .reshape(Lc, B, Hp)
        for c in range(_NSPLIT)
    ]
    out_t = None
    for c in range(_NSPLIT):
        in_specs = [
            pl.BlockSpec((1, B, Hp), lambda l: (l, 0, 0)),
            pl.BlockSpec((Vo, Hp), lambda l: (0, 0)),
            pl.BlockSpec((Vo, 1), lambda l: (0, 0)),
        ]
        out_spec = pl.BlockSpec(
            (1, Vo, B), functools.partial(lambda c0, l: (c0 + l, 0, 0), c * Lc)
        )
        if c == 0:
            out_t = pl.pallas_call(
                _proj_first,
                grid=(Lc,),
                in_specs=in_specs,
                out_specs=out_spec,
                out_shape=out_shape,
            )(hidden[c], w_p, bias)
        else:
            out_t = pl.pallas_call(
                _proj_next,
                grid=(Lc,),
                in_specs=in_specs
                + [pl.BlockSpec(memory_space=pltpu.HBM)],
                out_specs=out_spec,
                out_shape=out_shape,
                input_output_aliases={3: 0},
            )(hidden[c], w_p, bias, out_t)
    return jnp.transpose(out_t, (2, 0, 1))


# speedup vs baseline: 5.1612x; 1.0115x over previous
import functools

import jax
import jax.numpy as jnp
from jax import lax
from jax.experimental import pallas as pl
from jax.experimental.pallas import tpu as pltpu
from jax.experimental.pallas import tpu_sc as plsc

_NSPLIT = 5


@functools.cache
def _gather_fn(V, Hp, N):
    info = plsc.get_sparse_core_info()
    nw = info.num_cores * info.num_subcores
    rows_per_w = N // nw
    ch = 80
    nch = rows_per_w // ch
    mesh = plsc.VectorSubcoreMesh(core_axis_name="c", subcore_axis_name="s")

    @functools.partial(
        pl.kernel, mesh=mesh,
        out_type=jax.ShapeDtypeStruct((N, Hp), jnp.float32),
        scratch_types=[
            pltpu.VMEM((rows_per_w,), jnp.int32),
            pltpu.VMEM((ch, Hp), jnp.float32),
            pltpu.SemaphoreType.DMA,
        ],
    )
    def k(table_hbm, idx_hbm, out_hbm, idx_v, rows_v, sem):
        wid = lax.axis_index("s") * info.num_cores + lax.axis_index("c")
        base = wid * rows_per_w
        pltpu.sync_copy(idx_hbm.at[pl.ds(base, rows_per_w)], idx_v)

        def body(c, carry):
            pltpu.async_copy(
                table_hbm.at[idx_v.at[pl.ds(c * ch, ch)]], rows_v, sem
            ).wait()
            pltpu.sync_copy(rows_v, out_hbm.at[pl.ds(base + c * ch, ch)])
            return carry

        lax.fori_loop(0, nch, body, 0)

    return k


def _proj_first(h_ref, w_ref, b_ref, out_ref):
    out_ref[0] = lax.dot_general(
        w_ref[...], h_ref[0],
        dimension_numbers=(((1,), (1,)), ((), ())),
        preferred_element_type=jnp.float32,
    ) + b_ref[...]


def _proj_next(h_ref, w_ref, b_ref, carry_ref, out_ref):
    out_ref[0] = lax.dot_general(
        w_ref[...], h_ref[0],
        dimension_numbers=(((1,), (1,)), ((), ())),
        preferred_element_type=jnp.float32,
    ) + b_ref[...]


def kernel(input_ids, emb_table, lm_head_w, lm_head_b):
    V, H = emb_table.shape
    Vo = lm_head_w.shape[0]
    B, L = input_ids.shape
    Hp = 128
    Lc = L // _NSPLIT
    emb_p = jnp.pad(emb_table, ((0, 0), (0, Hp - H)))
    w_p = jnp.pad(lm_head_w, ((0, 0), (0, Hp - H)))
    bias = lm_head_b.reshape(Vo, 1)
    ids_t = input_ids.T.reshape(-1).astype(jnp.int32)
    gather = _gather_fn(V, Hp, B * Lc)
    out_shape = jax.ShapeDtypeStruct((L, Vo, B), jnp.float32)
    hidden = [
        gather(emb_p, lax.dynamic_slice_in_dim(ids_t, c * B * Lc, B * Lc))
        .reshape(Lc, B, Hp)
        for c in range(_NSPLIT)
    ]
    out_t = None
    for c in range(_NSPLIT):
        in_specs = [
            pl.BlockSpec((1, B, Hp), lambda l: (l, 0, 0)),
            pl.BlockSpec((Vo, Hp), lambda l: (0, 0)),
            pl.BlockSpec((Vo, 1), lambda l: (0, 0)),
        ]
        out_spec = pl.BlockSpec(
            (1, Vo, B), functools.partial(lambda c0, l: (c0 + l, 0, 0), c * Lc)
        )
        if c == 0:
            out_t = pl.pallas_call(
                _proj_first,
                grid=(Lc,),
                in_specs=in_specs,
                out_specs=out_spec,
                out_shape=out_shape,
            )(hidden[c], w_p, bias)
        else:
            out_t = pl.pallas_call(
                _proj_next,
                grid=(Lc,),
                in_specs=in_specs
                + [pl.BlockSpec(memory_space=pltpu.HBM)],
                out_specs=out_spec,
                out_shape=out_shape,
                input_output_aliases={3: 0},
            )(hidden[c], w_p, bias, out_t)
    return jnp.transpose(out_t, (2, 0, 1))


# speedup vs baseline: 5.8453x; 1.1325x over previous
import functools

import jax
import jax.numpy as jnp
from jax import lax
from jax.experimental import pallas as pl
from jax.experimental.pallas import tpu as pltpu
from jax.experimental.pallas import tpu_sc as plsc

_SPLITS = (5, 15, 30)


@functools.cache
def _gather_fn(V, Hp, N):
    info = plsc.get_sparse_core_info()
    nw = info.num_cores * info.num_subcores
    rows_per_w = N // nw
    ch = 80
    nch = rows_per_w // ch
    grp = max(d for d in (4, 3, 2, 1) if nch % d == 0)
    mesh = plsc.VectorSubcoreMesh(core_axis_name="c", subcore_axis_name="s")

    @functools.partial(
        pl.kernel, mesh=mesh,
        out_type=jax.ShapeDtypeStruct((N, Hp), jnp.float32),
        scratch_types=[
            pltpu.VMEM((rows_per_w,), jnp.int32),
            pltpu.VMEM((grp, ch, Hp), jnp.float32),
            pltpu.VMEM_SHARED((V, Hp), jnp.float32),
            pltpu.SemaphoreType.DMA,
            pltpu.SemaphoreType.DMA,
        ],
    )
    def k(table_hbm, idx_hbm, out_hbm, idx_v, rows_v, tab_s, gsem, ssem):
        wid = lax.axis_index("s") * info.num_cores + lax.axis_index("c")
        base = wid * rows_per_w
        @pl.when(lax.axis_index("s") == 0)
        def _():
            pltpu.sync_copy(table_hbm, tab_s)

        pltpu.sync_copy(idx_hbm.at[pl.ds(base, rows_per_w)], idx_v)
        plsc.subcore_barrier()

        def body(g, carry):
            c0 = g * grp
            gathers = []
            for j in range(grp):
                d = pltpu.make_async_copy(
                    tab_s.at[idx_v.at[pl.ds((c0 + j) * ch, ch)]],
                    rows_v.at[j], gsem,
                )
                d.start()
                gathers.append(d)
            for d in gathers:
                d.wait()
            stores = []
            for j in range(grp):
                d = pltpu.make_async_copy(
                    rows_v.at[j], out_hbm.at[pl.ds(base + (c0 + j) * ch, ch)],
                    ssem,
                )
                d.start()
                stores.append(d)
            for d in stores:
                d.wait()
            return carry

        lax.fori_loop(0, nch // grp, body, 0)

    return k


def _proj_first(h_ref, w_ref, b_ref, out_ref):
    h = h_ref[0][:, : w_ref.shape[1]]
    out_ref[0] = lax.dot_general(
        w_ref[...], h,
        dimension_numbers=(((1,), (1,)), ((), ())),
        preferred_element_type=jnp.float32,
    ) + b_ref[...]


def _proj_next(h_ref, w_ref, b_ref, carry_ref, out_ref):
    h = h_ref[0][:, : w_ref.shape[1]]
    out_ref[0] = lax.dot_general(
        w_ref[...], h,
        dimension_numbers=(((1,), (1,)), ((), ())),
        preferred_element_type=jnp.float32,
    ) + b_ref[...]


def kernel(input_ids, emb_table, lm_head_w, lm_head_b):
    V, H = emb_table.shape
    Vo = lm_head_w.shape[0]
    B, L = input_ids.shape
    Hp = 128
    emb_p = jnp.pad(emb_table, ((0, 0), (0, Hp - H)))
    bias = lm_head_b.reshape(Vo, 1)
    ids_t = input_ids.T.reshape(-1).astype(jnp.int32)
    out_shape = jax.ShapeDtypeStruct((L, Vo, B), jnp.float32)
    offs = [sum(_SPLITS[:c]) for c in range(len(_SPLITS))]
    hidden = [
        _gather_fn(V, Hp, B * Lc)(
            emb_p, lax.dynamic_slice_in_dim(ids_t, off * B, B * Lc)
        ).reshape(Lc, B, Hp)
        for c, (off, Lc) in enumerate(zip(offs, _SPLITS))
    ]
    out_t = None
    for c, (off, Lc) in enumerate(zip(offs, _SPLITS)):
        in_specs = [
            pl.BlockSpec((1, B, Hp), lambda l: (l, 0, 0)),
            pl.BlockSpec((Vo, H), lambda l: (0, 0)),
            pl.BlockSpec((Vo, 1), lambda l: (0, 0)),
        ]
        out_spec = pl.BlockSpec(
            (1, Vo, B), functools.partial(lambda c0, l: (c0 + l, 0, 0), off)
        )
        if c == 0:
            out_t = pl.pallas_call(
                _proj_first,
                grid=(Lc,),
                in_specs=in_specs,
                out_specs=out_spec,
                out_shape=out_shape,
            )(hidden[c], lm_head_w, bias)
        else:
            out_t = pl.pallas_call(
                _proj_next,
                grid=(Lc,),
                in_specs=in_specs
                + [pl.BlockSpec(memory_space=pltpu.HBM)],
                out_specs=out_spec,
                out_shape=out_shape,
                input_output_aliases={3: 0},
            )(hidden[c], lm_head_w, bias, out_t)
    return jnp.transpose(out_t, (2, 0, 1))
